# baseline (speedup 1.0000x reference)
import jax
import jax.numpy as jnp
from jax import lax
from jax.experimental import pallas as pl
from jax.experimental.pallas import tpu as pltpu
from jax.experimental.pallas import tpu_sc as plsc

B = 4096
C = 4096
T = 32
Q = 2 * T
P = 128
NC = 2
NS = 16
NW = NC * NS
S = 2560
ROWS_PER_W = S // NW
NBUF = 4
STEPS = ROWS_PER_W // NBUF
RB = 128


def _flip_body(x_hbm, out_hbm, ins, outs, slis, ssos):
    wid = lax.axis_index("s") * NC + lax.axis_index("c")
    row0 = wid * ROWS_PER_W

    def load(g, buf, sem):
        pltpu.async_copy(x_hbm.at[pl.ds(row0 + g, 1)], buf, sem)

    def store(g, buf, sem):
        pltpu.async_copy(buf, out_hbm.at[pl.ds(row0 + g, 1)], sem)

    def wait_load(buf, sem):
        pltpu.make_async_copy(x_hbm.at[pl.ds(0, 1)], buf, sem).wait()

    def wait_store(buf, sem):
        pltpu.make_async_copy(buf, out_hbm.at[pl.ds(0, 1)], sem).wait()

    def compute(in_v, out_v):
        @plsc.parallel_loop(0, Q, 1, unroll=4)
        def _(qo):
            qi = 62 - qo + 2 * (qo & 1)
            for w in range(8):
                vals = in_v[0, qi, pl.ds((7 - w) * 16, 16)]
                out_v[0, qo, pl.ds(w * 16, 16)] = lax.rev(vals, (0,))

    for b in range(NBUF):
        load(b, ins[b], slis[b])

    def step(i, _):
        for b in range(NBUF):
            g = NBUF * i + b
            wait_load(ins[b], slis[b])

            @pl.when(i > 0)
            def _():
                wait_store(outs[b], ssos[b])

            compute(ins[b], outs[b])
            store(g, outs[b], ssos[b])

            @pl.when(i < STEPS - 1)
            def _():
                load(g + NBUF, ins[b], slis[b])

        return 0

    lax.fori_loop(0, STEPS, step, 0)
    for b in range(NBUF):
        wait_store(outs[b], ssos[b])


def _flip_entry(x_hbm, out_hbm, *scratch):
    ins = scratch[0:NBUF]
    outs = scratch[NBUF:2 * NBUF]
    slis = scratch[2 * NBUF:3 * NBUF]
    ssos = scratch[3 * NBUF:4 * NBUF]
    _flip_body(x_hbm, out_hbm, ins, outs, slis, ssos)


def _sc_flip(xv):
    mesh = plsc.VectorSubcoreMesh(core_axis_name="c", subcore_axis_name="s")
    return pl.kernel(
        _flip_entry,
        out_type=jax.ShapeDtypeStruct((S, Q, P), jnp.float32),
        mesh=mesh,
        scratch_types=(
            [pltpu.VMEM((1, Q, P), jnp.float32)] * (2 * NBUF)
            + [pltpu.SemaphoreType.DMA] * (2 * NBUF)
        ),
        compiler_params=pltpu.CompilerParams(needs_layout_passes=False),
    )(xv)


def _tc_body(x_ref, o_ref):
    v = x_ref[...]
    idx = P - 1 - lax.broadcasted_iota(jnp.int32, (RB, 8, P), 2)
    v = jnp.take_along_axis(v, idx, axis=2)
    j = lax.broadcasted_iota(jnp.int32, (RB, 8, P), 1)
    sidx = 6 - j + 2 * (j & 1)
    o_ref[...] = jnp.take_along_axis(v, sidx, axis=1)


def _tc_flip(xv):
    nrb = (B - S) // RB
    return pl.pallas_call(
        _tc_body,
        grid=(nrb, Q // 8),
        in_specs=[pl.BlockSpec((RB, 8, P), lambda i, u: (i + S // RB, Q // 8 - 1 - u, 0))],
        out_specs=pl.BlockSpec((RB, 8, P), lambda i, u: (i, u, 0)),
        out_shape=jax.ShapeDtypeStruct((B - S, Q, P), jnp.float32),
    )(xv)


@jax.jit
def _flip(xv):
    y_sc = _sc_flip(xv)
    y_tc = _tc_flip(xv)
    return jnp.concatenate([y_sc, y_tc], axis=0)


def kernel(x, c):
    xv = x.reshape(B, T, P, 2).transpose(0, 1, 3, 2).reshape(B, Q, P)
    yv = _flip(xv)
    return yv.reshape(B, T, 2, P).transpose(0, 1, 3, 2).reshape(B, C, 2)


# speedup vs baseline: 1.9017x; 1.9017x over previous
import jax
import jax.numpy as jnp
from jax import lax
from jax.experimental import pallas as pl
from jax.experimental.pallas import tpu as pltpu
from jax.experimental.pallas import tpu_sc as plsc

B = 4096
C = 4096
T = 32
Q = 2 * T
P = 128
NC = 2
NS = 16
NW = NC * NS
ROWS_PER_W = B // NW
NBUF = 4
STEPS = ROWS_PER_W // NBUF


def _flip_body(x_hbm, out_hbm, ins, outs, slis, ssos):
    wid = lax.axis_index("s") * NC + lax.axis_index("c")
    row0 = wid * ROWS_PER_W

    def load(g, buf, sem):
        pltpu.async_copy(x_hbm.at[pl.ds(row0 + g, 1)], buf, sem)

    def store(g, buf, sem):
        pltpu.async_copy(buf, out_hbm.at[pl.ds(row0 + g, 1)], sem)

    def wait_load(buf, sem):
        pltpu.make_async_copy(x_hbm.at[pl.ds(0, 1)], buf, sem).wait()

    def wait_store(buf, sem):
        pltpu.make_async_copy(buf, out_hbm.at[pl.ds(0, 1)], sem).wait()

    def compute(in_v, out_v):
        @plsc.parallel_loop(0, Q, 1, unroll=4)
        def _(qo):
            qi = 62 - qo + 2 * (qo & 1)
            for w in range(8):
                vals = in_v[0, qi, pl.ds((7 - w) * 16, 16)]
                out_v[0, qo, pl.ds(w * 16, 16)] = lax.rev(vals, (0,))

    for b in range(NBUF):
        load(b, ins[b], slis[b])

    def step(i, _):
        for b in range(NBUF):
            g = NBUF * i + b
            wait_load(ins[b], slis[b])

            @pl.when(i > 0)
            def _():
                wait_store(outs[b], ssos[b])

            compute(ins[b], outs[b])
            store(g, outs[b], ssos[b])

            @pl.when(i < STEPS - 1)
            def _():
                load(g + NBUF, ins[b], slis[b])

        return 0

    lax.fori_loop(0, STEPS, step, 0)
    for b in range(NBUF):
        wait_store(outs[b], ssos[b])


def _flip_entry(x_hbm, out_hbm, *scratch):
    ins = scratch[0:NBUF]
    outs = scratch[NBUF:2 * NBUF]
    slis = scratch[2 * NBUF:3 * NBUF]
    ssos = scratch[3 * NBUF:4 * NBUF]
    _flip_body(x_hbm, out_hbm, ins, outs, slis, ssos)


@jax.jit
def _flip(xv):
    mesh = plsc.VectorSubcoreMesh(core_axis_name="c", subcore_axis_name="s")
    return pl.kernel(
        _flip_entry,
        out_type=jax.ShapeDtypeStruct((B, Q, P), jnp.float32),
        mesh=mesh,
        scratch_types=(
            [pltpu.VMEM((1, Q, P), jnp.float32)] * (2 * NBUF)
            + [pltpu.SemaphoreType.DMA] * (2 * NBUF)
        ),
        compiler_params=pltpu.CompilerParams(needs_layout_passes=False),
    )(xv)


def kernel(x, c):
    xv = x.reshape(B, T, P, 2).transpose(0, 1, 3, 2).reshape(B, Q, P)
    yv = _flip(xv)
    return yv.reshape(B, T, 2, P).transpose(0, 1, 3, 2).reshape(B, C, 2)
